# baseline (speedup 1.0000x reference)
import functools

import jax
import jax.numpy as jnp
from jax import lax
from jax.experimental import pallas as pl
from jax.experimental.pallas import tpu as pltpu
from jax.experimental.pallas import tpu_sc as plsc

K = 24
D = 32
TW = 128
SB = 8


def _knn_mlp_body(feat_ref, xyzr_ref, xyzt_ref, w1_ref, b1_ref, w2_ref,
                  b2_ref, idx_ref, tab_ref, key_ref):
    R = feat_ref.shape[1]
    Nn = xyzt_ref.shape[2]
    b = pl.program_id(0)

    f = feat_ref[0]
    xr = xyzr_ref[0]
    xt = xyzt_ref[0]

    cross = jnp.dot(xr, xt, preferred_element_type=jnp.float32)
    rn = jnp.sum(xr * xr, axis=1, keepdims=True)
    cn = jnp.sum(xt * xt, axis=0, keepdims=True)
    dist = jnp.maximum(rn - 2.0 * cross + cn, 0.0)

    h1 = jnp.maximum(
        jnp.dot(f, w1_ref[...], preferred_element_type=jnp.float32)
        + b1_ref[0][None, :], 0.0)
    hh = jnp.maximum(
        jnp.dot(h1, w2_ref[...], preferred_element_type=jnp.float32)
        + b2_ref[0][None, :], 0.0)

    tab_ref[...] = jnp.concatenate(
        [xr, jnp.zeros((R, 13), jnp.float32), hh,
         jnp.zeros((R, TW - 48), jnp.float32)], axis=1)

    iota = lax.broadcasted_iota(jnp.int32, (R, Nn), 1)
    kio = lax.broadcasted_iota(jnp.int32, (SB, K), 1)
    key_ref[...] = (lax.bitcast_convert_type(dist, jnp.int32)
                    & jnp.int32(-2048)) | iota
    big = jnp.int32(0x7FFFFFFF)
    gs = 2 * SB

    def group(g, carry):
        outs = []
        for half in range(2):
            kcur = key_ref[pl.ds(g * gs + half * SB, SB)]
            acc = jnp.zeros((SB, K), jnp.int32)
            for t in range(K):
                mm = jnp.min(kcur, axis=1, keepdims=True)
                acc = jnp.where(kio == t, mm & 2047, acc)
                kcur = jnp.where(kcur == mm, big, kcur)
            outs.append(acc)
        idx_ref[0, pl.ds(g * gs, gs), :] = (
            jnp.concatenate(outs, axis=0) + b * Nn)
        return carry

    lax.fori_loop(0, R // gs, group, 0)


def _agg_body(g_ref, xyz_ref, wr_ref, ws_ref, bs_ref, out_ref):
    P = xyz_ref.shape[0]
    DD = D * D

    g = g_ref[...]
    gx = g[:, :, 0:3]
    gh = jnp.reshape(g[:, :, 16:16 + D], (K * P, D))
    x = xyz_ref[...]
    rel = jnp.reshape(gx - x[None, :, :], (K * P, 3))

    rel4 = jnp.concatenate([rel, jnp.ones((K * P, 1), jnp.float32)], axis=1)
    w = jnp.dot(rel4, wr_ref[...],
                preferred_element_type=jnp.float32)
    w4 = jnp.reshape(w, (K, P, DD))

    mx = jnp.max(jnp.max(w4, axis=0), axis=1, keepdims=True)
    e = jnp.exp(w4 - mx[None, :, :])

    ci = lax.broadcasted_iota(jnp.int32, (D, DD), 1)
    ri = lax.broadcasted_iota(jnp.int32, (D, DD), 0)
    qsel = (ci // D == ri).astype(jnp.float32)
    rsel = (ci % D == ri).astype(jnp.float32)

    hb = jnp.dot(gh, qsel, preferred_element_type=jnp.float32)
    m = e * jnp.reshape(hb, (K, P, DD))

    s1 = jnp.sum(e, axis=0)
    n1 = jnp.sum(m, axis=0)
    den = lax.dot_general(s1, rsel, (((1,), (1,)), ((), ())),
                          preferred_element_type=jnp.float32)
    num = lax.dot_general(n1, rsel, (((1,), (1,)), ((), ())),
                          preferred_element_type=jnp.float32)

    o = num / den
    out_ref[...] = jnp.dot(o, ws_ref[...],
                           preferred_element_type=jnp.float32) \
        + bs_ref[0][None, :]


def _sc_gather(table, idxf):
    nw = 32
    total = idxf.shape[0]
    bpw = total // nw
    nchunk = 4
    cb = bpw // nchunk
    mesh = plsc.VectorSubcoreMesh(core_axis_name="c", subcore_axis_name="s")

    @functools.partial(
        pl.kernel, mesh=mesh,
        out_type=jax.ShapeDtypeStruct((total, TW), jnp.float32),
        scratch_types=[
            pltpu.VMEM((cb,), jnp.int32),
            pltpu.VMEM((cb, TW), jnp.float32),
            pltpu.SemaphoreType.DMA,
        ],
    )
    def gk(table_hbm, idx_hbm, out_hbm, idx_v, rows_v, sem):
        wid = lax.axis_index("s") * 2 + lax.axis_index("c")
        base = wid * bpw
        for c in range(nchunk):
            off = base + c * cb
            pltpu.sync_copy(idx_hbm.at[pl.ds(off, cb)], idx_v)
            pltpu.async_copy(table_hbm.at[idx_v], rows_v, sem).wait()
            pltpu.sync_copy(rows_v, out_hbm.at[pl.ds(off, cb)])

    return gk(table, idxf)


def kernel(feature, xyz, knn_num, W1, b1, W2, b2, Wr, br, Ws, bs):
    Bd, Nn, IN = feature.shape
    R = 64
    P = 64
    BN = Bd * Nn

    xyzt = jnp.swapaxes(xyz, 1, 2)

    idxg, table = pl.pallas_call(
        _knn_mlp_body,
        grid=(Bd, Nn // R),
        in_specs=[
            pl.BlockSpec((1, R, IN), lambda b, r: (b, r, 0)),
            pl.BlockSpec((1, R, 3), lambda b, r: (b, r, 0)),
            pl.BlockSpec((1, 3, Nn), lambda b, r: (b, 0, 0)),
            pl.BlockSpec((IN, D), lambda b, r: (0, 0)),
            pl.BlockSpec((1, D), lambda b, r: (0, 0)),
            pl.BlockSpec((D, D), lambda b, r: (0, 0)),
            pl.BlockSpec((1, D), lambda b, r: (0, 0)),
        ],
        out_specs=[
            pl.BlockSpec((1, R, K), lambda b, r: (b, r, 0)),
            pl.BlockSpec((R, TW), lambda b, r: (b * (Nn // R) + r, 0)),
        ],
        out_shape=[
            jax.ShapeDtypeStruct((Bd, Nn, K), jnp.int32),
            jax.ShapeDtypeStruct((BN, TW), jnp.float32),
        ],
        scratch_shapes=[pltpu.VMEM((R, Nn), jnp.int32)],
    )(feature, xyz, xyzt, W1, b1.reshape(1, D), W2, b2.reshape(1, D))

    idxf = jnp.transpose(idxg.reshape(BN, K)).reshape(-1)
    gathered = _sc_gather(table, idxf).reshape(K, BN, TW)

    wr4 = jnp.concatenate([Wr, br[None, :]], axis=0)

    out_flat = pl.pallas_call(
        _agg_body,
        grid=(BN // P,),
        in_specs=[
            pl.BlockSpec((K, P, TW), lambda p: (0, p, 0)),
            pl.BlockSpec((P, 3), lambda p: (p, 0)),
            pl.BlockSpec((4, D * D), lambda p: (0, 0)),
            pl.BlockSpec((D, D), lambda p: (0, 0)),
            pl.BlockSpec((1, D), lambda p: (0, 0)),
        ],
        out_specs=pl.BlockSpec((P, D), lambda p: (p, 0)),
        out_shape=jax.ShapeDtypeStruct((BN, D), jnp.float32),
    )(gathered, xyz.reshape(BN, 3), wr4, Ws, bs.reshape(1, D))

    out = out_flat.reshape(Bd, Nn, D)
    out = out + (jnp.asarray(knn_num, out.dtype) - jnp.float32(K))
    return (out, Nn)


# speedup vs baseline: 3.7322x; 3.7322x over previous
import functools

import jax
import jax.numpy as jnp
from jax import lax
from jax.experimental import pallas as pl
from jax.experimental.pallas import tpu as pltpu
from jax.experimental.pallas import tpu_sc as plsc

K = 24
D = 32
TW = 128
SB = 16


def _knn_mlp_body(feat_ref, xyzr_ref, xyzt_ref, w1_ref, b1_ref, w2_ref,
                  b2_ref, idx_ref, tab_ref):
    R = feat_ref.shape[0]
    Nn = xyzt_ref.shape[1]

    f = feat_ref[...]
    xr = xyzr_ref[...]
    xt = xyzt_ref[...]

    cross = jnp.dot(xr, xt, preferred_element_type=jnp.float32)
    rn = jnp.sum(xr * xr, axis=1, keepdims=True)
    cn = jnp.sum(xt * xt, axis=0, keepdims=True)
    dist = jnp.maximum(rn - 2.0 * cross + cn, 0.0)

    h1 = jnp.maximum(
        jnp.dot(f, w1_ref[...], preferred_element_type=jnp.float32)
        + b1_ref[0][None, :], 0.0)
    hh = jnp.maximum(
        jnp.dot(h1, w2_ref[...], preferred_element_type=jnp.float32)
        + b2_ref[0][None, :], 0.0)

    tab_ref[...] = jnp.concatenate(
        [xr, jnp.zeros((R, 13), jnp.float32), hh,
         jnp.zeros((R, TW - 48), jnp.float32)], axis=1)

    iota = lax.broadcasted_iota(jnp.int32, (R, Nn), 1)
    kio = lax.broadcasted_iota(jnp.int32, (SB, K), 1)
    key0 = (lax.bitcast_convert_type(dist, jnp.int32)
            & jnp.int32(-2048)) | iota
    big = jnp.int32(0x7FFFFFFF)

    outs = []
    for s in range(R // SB):
        kcur = key0[s * SB:(s + 1) * SB]
        acc = jnp.zeros((SB, K), jnp.int32)
        for t in range(K):
            mm = jnp.min(kcur, axis=1, keepdims=True)
            acc = jnp.where(kio == t, mm & 2047, acc)
            kcur = jnp.where(kcur == mm, big, kcur)
        outs.append(acc)
    idx_ref[...] = jnp.transpose(jnp.concatenate(outs, axis=0))


def _agg_body(g_ref, xyz_ref, wr_ref, ws_ref, bs_ref, out_ref):
    P = xyz_ref.shape[0]
    DD = D * D

    g = g_ref[...]
    gx = g[:, :, 0:3]
    gh = jnp.reshape(g[:, :, 16:16 + D], (K * P, D))
    x = xyz_ref[...]
    rel = jnp.reshape(gx - x[None, :, :], (K * P, 3))

    rel4 = jnp.concatenate([rel, jnp.ones((K * P, 1), jnp.float32)], axis=1)
    w = jnp.dot(rel4, wr_ref[...],
                preferred_element_type=jnp.float32)
    w4 = jnp.reshape(w, (K, P, DD))

    mx = jnp.max(jnp.max(w4, axis=0), axis=1, keepdims=True)
    e = jnp.exp(w4 - mx[None, :, :])

    ci = lax.broadcasted_iota(jnp.int32, (D, DD), 1)
    ri = lax.broadcasted_iota(jnp.int32, (D, DD), 0)
    qsel = (ci // D == ri).astype(jnp.float32)
    rsel = (ci % D == ri).astype(jnp.float32)

    hb = jnp.dot(gh, qsel, preferred_element_type=jnp.float32)
    m = e * jnp.reshape(hb, (K, P, DD))

    s1 = jnp.sum(e, axis=0)
    n1 = jnp.sum(m, axis=0)
    den = lax.dot_general(s1, rsel, (((1,), (1,)), ((), ())),
                          preferred_element_type=jnp.float32)
    num = lax.dot_general(n1, rsel, (((1,), (1,)), ((), ())),
                          preferred_element_type=jnp.float32)

    o = num / den
    out_ref[...] = jnp.dot(o, ws_ref[...],
                           preferred_element_type=jnp.float32) \
        + bs_ref[0][None, :]


def _sc_gather(table, idxf):
    nw = 32
    total = idxf.shape[0]
    bpw = total // nw
    nchunk = 8
    cb = bpw // nchunk
    mesh = plsc.VectorSubcoreMesh(core_axis_name="c", subcore_axis_name="s")

    @functools.partial(
        pl.kernel, mesh=mesh,
        out_type=jax.ShapeDtypeStruct((total, TW), jnp.float32),
        scratch_types=[
            pltpu.VMEM((cb,), jnp.int32),
            pltpu.VMEM((cb,), jnp.int32),
            pltpu.VMEM((cb, TW), jnp.float32),
            pltpu.VMEM((cb, TW), jnp.float32),
            pltpu.SemaphoreType.DMA,
            pltpu.SemaphoreType.DMA,
            pltpu.SemaphoreType.DMA,
            pltpu.SemaphoreType.DMA,
        ],
    )
    def gk(table_hbm, idx_hbm, out_hbm, i0, i1, r0, r1, sg0, sg1, sw0, sw1):
        wid = lax.axis_index("s") * 2 + lax.axis_index("c")
        base = wid * bpw
        ibuf, rbuf = [i0, i1], [r0, r1]
        sg, sw = [sg0, sg1], [sw0, sw1]
        gh = [None, None]
        wh = [None, None]
        pltpu.sync_copy(idx_hbm.at[pl.ds(base, cb)], i0)
        gh[0] = pltpu.async_copy(table_hbm.at[i0], r0, sg0)
        for c in range(nchunk):
            buf = c & 1
            nb = 1 - buf
            if c + 1 < nchunk:
                pltpu.sync_copy(
                    idx_hbm.at[pl.ds(base + (c + 1) * cb, cb)], ibuf[nb])
            gh[buf].wait()
            wh[buf] = pltpu.async_copy(
                rbuf[buf], out_hbm.at[pl.ds(base + c * cb, cb)], sw[buf])
            if c + 1 < nchunk:
                if wh[nb] is not None:
                    wh[nb].wait()
                gh[nb] = pltpu.async_copy(
                    table_hbm.at[ibuf[nb]], rbuf[nb], sg[nb])
        wh[0].wait()
        wh[1].wait()

    return gk(table, idxf)


def kernel(feature, xyz, knn_num, W1, b1, W2, b2, Wr, br, Ws, bs):
    Bd, Nn, IN = feature.shape
    R = 512
    P = 128

    wr4 = jnp.concatenate([Wr, br[None, :]], axis=0)

    stage1 = []
    for b in range(Bd):
        xyzt_b = jnp.swapaxes(xyz[b], 0, 1)
        stage1.append(pl.pallas_call(
            _knn_mlp_body,
            grid=(Nn // R,),
            in_specs=[
                pl.BlockSpec((R, IN), lambda r: (r, 0)),
                pl.BlockSpec((R, 3), lambda r: (r, 0)),
                pl.BlockSpec((3, Nn), lambda r: (0, 0)),
                pl.BlockSpec((IN, D), lambda r: (0, 0)),
                pl.BlockSpec((1, D), lambda r: (0, 0)),
                pl.BlockSpec((D, D), lambda r: (0, 0)),
                pl.BlockSpec((1, D), lambda r: (0, 0)),
            ],
            out_specs=[
                pl.BlockSpec((K, R), lambda r: (0, r)),
                pl.BlockSpec((R, TW), lambda r: (r, 0)),
            ],
            out_shape=[
                jax.ShapeDtypeStruct((K, Nn), jnp.int32),
                jax.ShapeDtypeStruct((Nn, TW), jnp.float32),
            ],
        )(feature[b], xyz[b], xyzt_b, W1, b1.reshape(1, D),
          W2, b2.reshape(1, D)))

    gathered = [
        _sc_gather(table_b, idx_b.reshape(-1)).reshape(K, Nn, TW)
        for idx_b, table_b in stage1]

    outs = []
    for b in range(Bd):
        outs.append(pl.pallas_call(
            _agg_body,
            grid=(Nn // P,),
            in_specs=[
                pl.BlockSpec((K, P, TW), lambda p: (0, p, 0)),
                pl.BlockSpec((P, 3), lambda p: (p, 0)),
                pl.BlockSpec((4, D * D), lambda p: (0, 0)),
                pl.BlockSpec((D, D), lambda p: (0, 0)),
                pl.BlockSpec((1, D), lambda p: (0, 0)),
            ],
            out_specs=pl.BlockSpec((P, D), lambda p: (p, 0)),
            out_shape=jax.ShapeDtypeStruct((Nn, D), jnp.float32),
        )(gathered[b], xyz[b], wr4, Ws, bs.reshape(1, D)))

    out = jnp.stack(outs, axis=0)
    out = out + (jnp.asarray(knn_num, out.dtype) - jnp.float32(K))
    return (out, Nn)
